# baseline (speedup 1.0000x reference)
import functools

import jax
import jax.numpy as jnp
from jax import lax
from jax.experimental import pallas as pl
from jax.experimental.pallas import tpu as pltpu
from jax.experimental.pallas import tpu_sc as plsc

_NP = 4


def kernel(labels, embedding_table):
    (B,) = labels.shape
    V, D = embedding_table.shape
    n_groups = (V - 1) // 8
    table3 = embedding_table[: n_groups * 8].reshape(n_groups, 8, D)
    null_row = embedding_table[V - 1]

    info = plsc.get_sparse_core_info()
    num_workers = info.num_cores * info.num_subcores
    b_per_w = B // num_workers
    L = info.num_lanes
    n_grp = b_per_w // L
    mesh = plsc.VectorSubcoreMesh(core_axis_name="c", subcore_axis_name="s")

    @functools.partial(
        pl.kernel,
        mesh=mesh,
        out_type=jax.ShapeDtypeStruct((B, D), jnp.float32),
        compiler_params=pltpu.CompilerParams(use_tc_tiling_on_sc=True),
        scratch_types=(
            [pltpu.VMEM((4, 128), jnp.int32),
             pltpu.VMEM((b_per_w,), jnp.int32),
             pltpu.VMEM((b_per_w,), jnp.int32),
             pltpu.VMEM((_NP * L, 8, D), jnp.float32),
             pltpu.VMEM((D,), jnp.float32)]
            + [pltpu.VMEM((L, D), jnp.float32) for _ in range(_NP)]
            + [pltpu.SemaphoreType.DMA] * (2 * _NP)
        ),
    )
    def _embed(labels_hbm, table_hbm, null_hbm, out_hbm,
               lab_v, k_v, renc_v, rows_v, null_v, *rest):
        out_bufs = rest[:_NP]
        gsem = rest[_NP:2 * _NP]
        ssem = rest[2 * _NP:]
        wid = lax.axis_index("s") * info.num_cores + lax.axis_index("c")
        base = wid * b_per_w
        pltpu.sync_copy(null_hbm, null_v)
        for c in range(4):
            pltpu.sync_copy(labels_hbm.at[pl.ds(base + c * 128, 128)],
                            lab_v.at[c])
        for i in range(b_per_w // L):
            c, off = i // 8, (i % 8) * L
            s = lab_v[c, pl.ds(off, L)]
            sel = jnp.where(s < 0, s + V, s)
            sel = jnp.minimum(jnp.maximum(sel, 0), V - 1)
            renc = (sel & 7) + jnp.where(sel == V - 1, 16, 0)
            k = jnp.minimum(sel >> 3, n_groups - 1)
            k_v[pl.ds(i * L, L)] = k
            renc_v[pl.ds(i * L, L)] = renc

        def fire(g, p):
            kv = k_v[pl.ds(g * L, L)]
            for l in range(L):
                pltpu.async_copy(table_hbm.at[pl.ds(kv[l], 1)],
                                 rows_v.at[pl.ds(p * L + l, 1)], gsem[p])

        def drain_gather(p):
            for _ in range(L):
                pltpu.make_async_copy(table_hbm.at[pl.ds(0, 1)],
                                      rows_v.at[pl.ds(0, 1)], gsem[p]).wait()

        def extract_store(g, p, drain_store_first):
            if drain_store_first:
                pltpu.make_async_copy(out_bufs[p], out_hbm.at[pl.ds(0, L)],
                                      ssem[p]).wait()
            rv = renc_v[pl.ds(g * L, L)]
            for l in range(L):
                re = rv[l]
                r = re & 7
                mv = jnp.broadcast_to(jnp.where(re >= 16, 1.0, 0.0), (L,))
                for q in range(D // L):
                    d = rows_v[p * L + l, r, pl.ds(q * L, L)]
                    n = null_v[pl.ds(q * L, L)]
                    out_bufs[p][l, pl.ds(q * L, L)] = d + (n - d) * mv
            pltpu.async_copy(out_bufs[p], out_hbm.at[pl.ds(base + g * L, L)],
                             ssem[p])

        for g in range(3):
            fire(g, g)

        def body(t, _):
            for u in range(4):
                g = 3 + 4 * t + u
                p = (3 + u) % 4
                pd = u % 4
                fire(g, p)
                drain_gather(pd)
                extract_store(g - 3, pd, True)
            return _

        for u in range(4):
            g = 3 + u
            p = (3 + u) % 4
            pd = u % 4
            fire(g, p)
            drain_gather(pd)
            extract_store(g - 3, pd, False)
        lax.fori_loop(1, 7, body, None)
        fire(31, 31 % 4)
        for u in range(4):
            g = 28 + u
            pd = g % 4
            drain_gather(pd)
            extract_store(g, pd, True)
        for p in range(4):
            pltpu.make_async_copy(out_bufs[p], out_hbm.at[pl.ds(0, L)],
                                  ssem[p]).wait()

    return _embed(labels.astype(jnp.int32), table3, null_row)
